# Initial kernel scaffold; baseline (speedup 1.0000x reference)
import functools

import jax
import jax.numpy as jnp
from jax import lax
from jax.experimental import pallas as pl
from jax.experimental.pallas import tpu as pltpu
from jax.experimental.pallas import tpu_sc as plsc

N_CLS = 100
CTX = 77
D = 512
B = 8


def _sc_broadcast(token_prefix, token_suffix, tok_flat):
    info = plsc.get_sparse_core_info()
    nc, ns = info.num_cores, info.num_subcores
    nw = nc * ns

    mesh = plsc.VectorSubcoreMesh(core_axis_name="c", subcore_axis_name="s")

    @functools.partial(
        pl.kernel,
        mesh=mesh,
        out_type=(
            jax.ShapeDtypeStruct((B * N_CLS, CTX, D), jnp.float32),
            jax.ShapeDtypeStruct((B, N_CLS * CTX), jnp.int32),
        ),
        scratch_types=[
            pltpu.VMEM((CTX, D), jnp.float32),
            pltpu.VMEM((N_CLS * CTX,), jnp.int32),
        ],
    )
    def k(prefix_hbm, suffix_hbm, tok_hbm, out_p, out_t, buf, tokbuf):
        wid = lax.axis_index("s") * nc + lax.axis_index("c")

        for c_loc in range((N_CLS + nw - 1) // nw):
            c = c_loc * nw + wid

            @pl.when(c < N_CLS)
            def _():
                pltpu.sync_copy(prefix_hbm.at[c], buf.at[pl.ds(0, 1)])
                pltpu.sync_copy(suffix_hbm.at[c], buf.at[pl.ds(1, CTX - 1)])
                for b in range(B):
                    pltpu.sync_copy(buf, out_p.at[b * N_CLS + c])

        @pl.when(jnp.logical_and(wid >= 8, wid < 8 + B))
        def _():
            pltpu.sync_copy(tok_hbm, tokbuf)
            pltpu.sync_copy(tokbuf, out_t.at[wid - 8])

    return k(token_prefix, token_suffix, tok_flat)


def kernel(indices, token_prefix, token_suffix, tokenized_prompts):
    del indices
    tok_flat = tokenized_prompts.reshape(N_CLS * CTX)
    prompts, tok = _sc_broadcast(token_prefix, token_suffix, tok_flat)
    return prompts, tok.reshape(B * N_CLS, CTX)


# speedup vs baseline: 1.2507x; 1.2507x over previous
import functools

import jax
import jax.numpy as jnp
from jax import lax
from jax.experimental import pallas as pl
from jax.experimental.pallas import tpu as pltpu
from jax.experimental.pallas import tpu_sc as plsc

N_CLS = 100
CTX = 77
D = 512
B = 8
LANES = 16


def _sc_broadcast(token_prefix, token_suffix, tokenized_prompts):
    info = plsc.get_sparse_core_info()
    nc, ns = info.num_cores, info.num_subcores
    nw = nc * ns

    mesh = plsc.VectorSubcoreMesh(core_axis_name="c", subcore_axis_name="s")

    @functools.partial(
        pl.kernel,
        mesh=mesh,
        out_type=(
            jax.ShapeDtypeStruct((B * N_CLS, CTX, D), jnp.float32),
            jax.ShapeDtypeStruct((B, N_CLS, CTX), jnp.int32),
        ),
        scratch_types=[
            pltpu.VMEM((CTX, D), jnp.float32),
            pltpu.VMEM((CTX - 1, D), jnp.float32),
            pltpu.VMEM((N_CLS, CTX), jnp.int32),
        ],
    )
    def k(prefix_hbm, suffix_hbm, tok_hbm, out_p, out_t, buf, bufs, tokbuf):
        wid = lax.axis_index("s") * nc + lax.axis_index("c")

        for c_loc in range((N_CLS + nw - 1) // nw):
            c = c_loc * nw + wid

            @pl.when(c < N_CLS)
            def _():
                pltpu.sync_copy(prefix_hbm.at[c], buf.at[pl.ds(0, 1)])
                pltpu.sync_copy(suffix_hbm.at[c], bufs)

                def shift_row(r, carry):
                    for kk in range(D // LANES):
                        sl = pl.ds(kk * LANES, LANES)
                        buf[r + 1, sl] = bufs[r, sl]
                    return carry

                lax.fori_loop(0, CTX - 1, shift_row, 0)

                for b in range(B):
                    pltpu.sync_copy(buf, out_p.at[b * N_CLS + c])

        @pl.when(jnp.logical_and(wid >= 8, wid < 8 + B))
        def _():
            pltpu.sync_copy(tok_hbm, tokbuf)
            pltpu.sync_copy(tokbuf, out_t.at[wid - 8])

    return k(token_prefix, token_suffix, tokenized_prompts)


def kernel(indices, token_prefix, token_suffix, tokenized_prompts):
    del indices
    prompts, tok = _sc_broadcast(token_prefix, token_suffix, tokenized_prompts)
    return prompts, tok.reshape(B * N_CLS, CTX)


# speedup vs baseline: 2.4368x; 1.9484x over previous
import functools

import jax
import jax.numpy as jnp
from jax import lax
from jax.experimental import pallas as pl
from jax.experimental.pallas import tpu as pltpu
from jax.experimental.pallas import tpu_sc as plsc

N_CLS = 100
CTX = 77
D = 512
B = 8
LANES = 16
T0 = 44


def _sc_prompts(prefix2, suffix_t):
    info = plsc.get_sparse_core_info()
    nc, ns = info.num_cores, info.num_subcores
    nw = nc * ns

    mesh = plsc.VectorSubcoreMesh(core_axis_name="c", subcore_axis_name="s")

    @functools.partial(
        pl.kernel,
        mesh=mesh,
        out_type=jax.ShapeDtypeStruct((CTX, B * N_CLS, D), jnp.float32),
        scratch_types=[
            pltpu.VMEM((2 * N_CLS + 4, D), jnp.float32),
        ],
    )
    def k(prefix_hbm, suffix_hbm, out_p, buf):
        wid = lax.axis_index("s") * nc + lax.axis_index("c")

        n_units = T0 * (B // 2)
        base, extra = n_units // nw, n_units % nw
        u0 = base * wid + jnp.minimum(wid, extra)
        n_mine = base + jnp.where(wid < extra, 1, 0)

        def load_slab(t):
            @pl.when(t == 0)
            def _():
                pltpu.sync_copy(prefix_hbm.at[pl.ds(0, 96)], buf.at[pl.ds(0, 96)])
                pltpu.sync_copy(
                    prefix_hbm.at[pl.ds(96, 4)], buf.at[pl.ds(2 * N_CLS, 4)]
                )

            @pl.when(t > 0)
            def _():
                pltpu.sync_copy(
                    suffix_hbm.at[t - 1, pl.ds(0, 96)], buf.at[pl.ds(0, 96)]
                )
                pltpu.sync_copy(
                    suffix_hbm.at[t - 1, pl.ds(96, 4)], buf.at[pl.ds(2 * N_CLS, 4)]
                )

            for rr in range(4):
                for kk in range(D // LANES):
                    sl = pl.ds(kk * LANES, LANES)
                    buf[96 + rr, sl] = buf[2 * N_CLS + rr, sl]

            def dup_row(r2, carry):
                for kk in range(D // LANES):
                    sl = pl.ds(kk * LANES, LANES)
                    buf[N_CLS + r2, sl] = buf[r2, sl]
                return carry

            lax.fori_loop(0, N_CLS, dup_row, 0)

        for i in range(base + 1):
            u = u0 + i
            t = u // (B // 2)
            j = u % (B // 2)
            first_or_new_slab = jnp.logical_or(i == 0, j == 0)

            @pl.when(jnp.logical_and(i < n_mine, first_or_new_slab))
            def _():
                load_slab(t)

            @pl.when(i < n_mine)
            def _():
                pltpu.sync_copy(
                    buf.at[pl.ds(0, 2 * N_CLS)],
                    out_p.at[t, pl.ds(j * 2 * N_CLS, 2 * N_CLS)],
                )

    return k(prefix2, suffix_t)


def _tc_prompts(suffix_t, out_p):
    def body(s_ref, _, o_ref):
        x = s_ref[...]
        o_ref[...] = jnp.concatenate([x] * B, axis=1)

    return pl.pallas_call(
        body,
        grid=(CTX - T0,),
        in_specs=[
            pl.BlockSpec((1, N_CLS, D), lambda i: (T0 - 1 + i, 0, 0)),
            pl.BlockSpec(memory_space=pl.ANY),
        ],
        out_specs=pl.BlockSpec((1, B * N_CLS, D), lambda i: (T0 + i, 0, 0)),
        out_shape=jax.ShapeDtypeStruct((CTX, B * N_CLS, D), jnp.float32),
        input_output_aliases={1: 0},
    )(suffix_t, out_p)


def _tc_tok(tok_t):
    def body(x_ref, o_ref):
        x = x_ref[...]
        o_ref[...] = jnp.concatenate([x] * B, axis=1)

    return pl.pallas_call(
        body,
        out_shape=jax.ShapeDtypeStruct((CTX, B * N_CLS), jnp.int32),
    )(tok_t)


def kernel(indices, token_prefix, token_suffix, tokenized_prompts):
    del indices
    prefix2 = token_prefix.reshape(N_CLS, D)
    suffix_t = jnp.transpose(token_suffix, (1, 0, 2))
    tok_t = jnp.transpose(tokenized_prompts)
    out_p_t = _sc_prompts(prefix2, suffix_t)
    out_p_t = _tc_prompts(suffix_t, out_p_t)
    tok_out = _tc_tok(tok_t)
    return jnp.transpose(out_p_t, (1, 0, 2)), jnp.transpose(tok_out)
